# baseline (speedup 1.0000x reference)
import jax
import jax.numpy as jnp
from jax.experimental import pallas as pl

TEMP = 8.0
CHUNK = 128


def _attn_kernel(q_ref, k_ref, v_ref, bias_ref, o_ref):
    k = k_ref[0]
    v = v_ref[0]
    L = k.shape[0]
    for c in range(L // CHUNK):
        rows = pl.ds(c * CHUNK, CHUNK)
        q = q_ref[0, rows, :] * (1.0 / TEMP)
        bias = bias_ref[0, rows, :]
        s = jax.lax.dot_general(
            q, k, (((1,), (1,)), ((), ())),
            preferred_element_type=jnp.float32)
        sm = s + bias
        mx = jnp.max(sm, axis=-1, keepdims=True)
        mx = jnp.where(jnp.isfinite(mx), mx, 0.0)
        ex = jnp.exp(sm - mx)
        den = jnp.sum(ex, axis=-1, keepdims=True)
        r = jnp.where(den == 0.0, 0.0, 1.0 / den)
        p = ex * r
        o_ref[0, rows, :] = jax.lax.dot_general(
            p, v, (((1,), (0,)), ((), ())),
            preferred_element_type=jnp.float32)


def kernel(q, k, v, mask):
    B, H, L, d = q.shape
    q3 = q.reshape(B * H, L, d)
    k3 = k.reshape(B * H, L, d)
    v3 = v.reshape(B * H, L, d)
    bias = jnp.where(mask, 0.0, -jnp.inf).astype(jnp.float32)
    out = pl.pallas_call(
        _attn_kernel,
        grid=(B * H,),
        in_specs=[
            pl.BlockSpec((1, L, d), lambda i: (i, 0, 0)),
            pl.BlockSpec((1, L, d), lambda i: (i, 0, 0)),
            pl.BlockSpec((1, L, d), lambda i: (i, 0, 0)),
            pl.BlockSpec((1, L, L), lambda i: (i // H, 0, 0)),
        ],
        out_specs=pl.BlockSpec((1, L, d), lambda i: (i, 0, 0)),
        out_shape=jax.ShapeDtypeStruct((B * H, L, d), jnp.float32),
    )(q3, k3, v3, bias)
    return out.reshape(B, H, L, d)


# speedup vs baseline: 1.2511x; 1.2511x over previous
import jax
import jax.numpy as jnp
from jax.experimental import pallas as pl

TEMP = 8.0
CHUNK = 128


def _attn_kernel(q_ref, k_ref, v_ref, bias_ref, o_ref):
    k = k_ref[0]
    v = v_ref[0]
    q = q_ref[0] * (1.0 / TEMP)
    bias = bias_ref[0]
    s = jax.lax.dot_general(
        q, k, (((1,), (1,)), ((), ())),
        preferred_element_type=jnp.float32)
    sm = s + bias
    mx = jnp.max(sm, axis=-1, keepdims=True)
    mx = jnp.where(jnp.isfinite(mx), mx, 0.0)
    ex = jnp.exp(sm - mx)
    den = jnp.sum(ex, axis=-1, keepdims=True)
    r = jnp.where(den == 0.0, 0.0, 1.0 / den)
    p = ex * r
    o_ref[0] = jax.lax.dot_general(
        p, v, (((1,), (0,)), ((), ())),
        preferred_element_type=jnp.float32)


def kernel(q, k, v, mask):
    B, H, L, d = q.shape
    q3 = q.reshape(B * H, L, d)
    k3 = k.reshape(B * H, L, d)
    v3 = v.reshape(B * H, L, d)
    bias = jnp.where(mask, 0.0, -jnp.inf).astype(jnp.float32)
    out = pl.pallas_call(
        _attn_kernel,
        grid=(B * H,),
        in_specs=[
            pl.BlockSpec((1, L, d), lambda i: (i, 0, 0)),
            pl.BlockSpec((1, L, d), lambda i: (i, 0, 0)),
            pl.BlockSpec((1, L, d), lambda i: (i, 0, 0)),
            pl.BlockSpec((1, L, L), lambda i: (i // H, 0, 0)),
        ],
        out_specs=pl.BlockSpec((1, L, d), lambda i: (i, 0, 0)),
        out_shape=jax.ShapeDtypeStruct((B * H, L, d), jnp.float32),
    )(q3, k3, v3, bias)
    return out.reshape(B, H, L, d)


# speedup vs baseline: 2.1354x; 1.7068x over previous
import jax
import jax.numpy as jnp
from jax.experimental import pallas as pl

TEMP = 8.0


def _attn_kernel(qt_ref, kt_ref, vt_ref, bias_ref, ot_ref):
    qt = qt_ref[0] * (1.0 / TEMP)
    kt = kt_ref[0]
    vt = vt_ref[0]
    bias = bias_ref[0]
    s = jax.lax.dot_general(
        qt, kt, (((0,), (0,)), ((), ())),
        preferred_element_type=jnp.float32)
    sm = s + bias
    mx = jnp.max(sm, axis=-1, keepdims=True)
    mx = jnp.where(jnp.isfinite(mx), mx, 0.0)
    ex = jnp.exp(sm - mx)
    den = jnp.sum(ex, axis=-1, keepdims=True)
    r = jnp.where(den == 0.0, 0.0, 1.0 / den)
    p = ex * r
    ot_ref[0] = jax.lax.dot_general(
        vt, p, (((1,), (1,)), ((), ())),
        preferred_element_type=jnp.float32)


def kernel(q, k, v, mask):
    B, H, L, d = q.shape
    qt = jnp.transpose(q, (0, 1, 3, 2)).reshape(B * H, d, L)
    kt = jnp.transpose(k, (0, 1, 3, 2)).reshape(B * H, d, L)
    vt = jnp.transpose(v, (0, 1, 3, 2)).reshape(B * H, d, L)
    bias = jnp.where(mask, 0.0, -jnp.inf).astype(jnp.float32)
    ot = pl.pallas_call(
        _attn_kernel,
        grid=(B * H,),
        in_specs=[
            pl.BlockSpec((1, d, L), lambda i: (i, 0, 0)),
            pl.BlockSpec((1, d, L), lambda i: (i, 0, 0)),
            pl.BlockSpec((1, d, L), lambda i: (i, 0, 0)),
            pl.BlockSpec((1, L, L), lambda i: (i // H, 0, 0)),
        ],
        out_specs=pl.BlockSpec((1, d, L), lambda i: (i, 0, 0)),
        out_shape=jax.ShapeDtypeStruct((B * H, d, L), jnp.float32),
    )(qt, kt, vt, bias)
    return jnp.transpose(ot.reshape(B, H, d, L), (0, 1, 3, 2))


# speedup vs baseline: 2.2884x; 1.0717x over previous
import jax
import jax.numpy as jnp
from jax.experimental import pallas as pl

TEMP = 8.0


def _attn_kernel(qt_ref, kt_ref, vt_ref, m_ref, ot_ref):
    qt = qt_ref[0] * (1.0 / TEMP)
    kt = kt_ref[0]
    vt = vt_ref[0]
    keep = m_ref[0] != 0
    s = jax.lax.dot_general(
        qt, kt, (((0,), (0,)), ((), ())),
        preferred_element_type=jnp.float32)
    sm = jnp.where(keep, s, -jnp.inf)
    mx = jnp.max(sm, axis=-1, keepdims=True)
    mx = jnp.where(jnp.isfinite(mx), mx, 0.0)
    ex = jnp.exp(sm - mx)
    den = jnp.sum(ex, axis=-1, keepdims=True)
    r = jnp.where(den == 0.0, 0.0, 1.0 / den)
    o = jax.lax.dot_general(
        vt, ex, (((1,), (1,)), ((), ())),
        preferred_element_type=jnp.float32)
    ot_ref[0] = o * r.reshape(1, -1)


def kernel(q, k, v, mask):
    B, H, L, d = q.shape
    qt = jnp.transpose(q, (0, 1, 3, 2)).reshape(B * H, d, L)
    kt = jnp.transpose(k, (0, 1, 3, 2)).reshape(B * H, d, L)
    vt = jnp.transpose(v, (0, 1, 3, 2)).reshape(B * H, d, L)
    m8 = mask.view(jnp.int8)
    ot = pl.pallas_call(
        _attn_kernel,
        grid=(B * H,),
        in_specs=[
            pl.BlockSpec((1, d, L), lambda i: (i, 0, 0)),
            pl.BlockSpec((1, d, L), lambda i: (i, 0, 0)),
            pl.BlockSpec((1, d, L), lambda i: (i, 0, 0)),
            pl.BlockSpec((1, L, L), lambda i: (i // H, 0, 0)),
        ],
        out_specs=pl.BlockSpec((1, d, L), lambda i: (i, 0, 0)),
        out_shape=jax.ShapeDtypeStruct((B * H, d, L), jnp.float32),
    )(qt, kt, vt, m8)
    return jnp.transpose(ot.reshape(B, H, d, L), (0, 1, 3, 2))


# speedup vs baseline: 2.9421x; 1.2856x over previous
import jax
import jax.numpy as jnp
from jax.experimental import pallas as pl

TEMP = 8.0


def _attn_kernel(qt_ref, kt_ref, vt_ref, m_ref, ot_ref):
    keep = m_ref[0] != 0
    H = qt_ref.shape[1]
    for h in range(H):
        qt = qt_ref[0, h] * (1.0 / TEMP)
        kt = kt_ref[0, h]
        vt = vt_ref[0, h]
        s = jax.lax.dot_general(
            qt, kt, (((0,), (0,)), ((), ())),
            preferred_element_type=jnp.float32)
        sm = jnp.where(keep, s, -jnp.inf)
        mx = jnp.max(sm, axis=-1, keepdims=True)
        mx = jnp.where(jnp.isfinite(mx), mx, 0.0)
        ex = jnp.exp(sm - mx)
        den = jnp.sum(ex, axis=-1, keepdims=True)
        r = jnp.where(den == 0.0, 0.0, 1.0 / den)
        o = jax.lax.dot_general(
            vt, ex, (((1,), (1,)), ((), ())),
            preferred_element_type=jnp.float32)
        ot_ref[0, h] = o * r.reshape(1, -1)


def kernel(q, k, v, mask):
    B, H, L, d = q.shape
    qt = jnp.transpose(q, (0, 1, 3, 2))
    kt = jnp.transpose(k, (0, 1, 3, 2))
    vt = jnp.transpose(v, (0, 1, 3, 2))
    m8 = mask.view(jnp.int8)
    ot = pl.pallas_call(
        _attn_kernel,
        grid=(B,),
        in_specs=[
            pl.BlockSpec((1, H, d, L), lambda i: (i, 0, 0, 0)),
            pl.BlockSpec((1, H, d, L), lambda i: (i, 0, 0, 0)),
            pl.BlockSpec((1, H, d, L), lambda i: (i, 0, 0, 0)),
            pl.BlockSpec((1, L, L), lambda i: (i, 0, 0)),
        ],
        out_specs=pl.BlockSpec((1, H, d, L), lambda i: (i, 0, 0, 0)),
        out_shape=jax.ShapeDtypeStruct((B, H, d, L), jnp.float32),
    )(qt, kt, vt, m8)
    return jnp.transpose(ot, (0, 1, 3, 2))
